# baseline (speedup 1.0000x reference)
import jax
import jax.numpy as jnp
from jax.experimental import pallas as pl
from jax.experimental.pallas import tpu as pltpu


def _copy_kernel(u_ref, i_ref, c_ref, ou_ref, oi_ref, oc_ref,
                 sem_u, sem_i, sem_c):
    cu = pltpu.make_async_copy(u_ref, ou_ref, sem_u)
    ci = pltpu.make_async_copy(i_ref, oi_ref, sem_i)
    cc = pltpu.make_async_copy(c_ref, oc_ref, sem_c)
    cu.start()
    ci.start()
    cc.start()
    cu.wait()
    ci.wait()
    cc.wait()


def kernel(emb_user, emb_item, emb_category):
    outs = pl.pallas_call(
        _copy_kernel,
        out_shape=tuple(
            jax.ShapeDtypeStruct(x.shape, x.dtype)
            for x in (emb_user, emb_item, emb_category)
        ),
        in_specs=[pl.BlockSpec(memory_space=pl.ANY)] * 3,
        out_specs=[pl.BlockSpec(memory_space=pl.ANY)] * 3,
        scratch_shapes=[pltpu.SemaphoreType.DMA] * 3,
    )(emb_user, emb_item, emb_category)
    return outs


# speedup vs baseline: 1.0073x; 1.0073x over previous
import jax
import jax.numpy as jnp
from jax.experimental import pallas as pl
from jax.experimental.pallas import tpu as pltpu

_N_CHUNKS = 32


def _copy_kernel(u_ref, i_ref, c_ref, ou_ref, oi_ref, oc_ref,
                 sems, csem):
    n_rows = u_ref.shape[0]
    chunk = n_rows // _N_CHUNKS
    copies = []
    for t, (src, dst) in enumerate(((u_ref, ou_ref), (i_ref, oi_ref))):
        for k in range(_N_CHUNKS):
            lo = k * chunk
            sz = chunk if k < _N_CHUNKS - 1 else n_rows - lo
            cp = pltpu.make_async_copy(
                src.at[pl.ds(lo, sz)], dst.at[pl.ds(lo, sz)], sems.at[t, k])
            cp.start()
            copies.append(cp)
    cc = pltpu.make_async_copy(c_ref, oc_ref, csem)
    cc.start()
    copies.append(cc)
    for cp in copies:
        cp.wait()


def kernel(emb_user, emb_item, emb_category):
    outs = pl.pallas_call(
        _copy_kernel,
        out_shape=tuple(
            jax.ShapeDtypeStruct(x.shape, x.dtype)
            for x in (emb_user, emb_item, emb_category)
        ),
        in_specs=[pl.BlockSpec(memory_space=pl.ANY)] * 3,
        out_specs=[pl.BlockSpec(memory_space=pl.ANY)] * 3,
        scratch_shapes=[pltpu.SemaphoreType.DMA((2, _N_CHUNKS)),
                        pltpu.SemaphoreType.DMA],
    )(emb_user, emb_item, emb_category)
    return outs


# speedup vs baseline: 15.4805x; 15.3687x over previous
import jax
import jax.numpy as jnp
from jax.experimental import pallas as pl
from jax.experimental.pallas import tpu as pltpu

_BLOCK = 10000


def _copy_kernel(u_ref, i_ref, c_ref, ou_ref, oi_ref, oc_ref, csem):
    step = pl.program_id(0)

    @pl.when(step == 0)
    def _start_cat():
        pltpu.make_async_copy(c_ref, oc_ref, csem).start()

    ou_ref[...] = u_ref[...]
    oi_ref[...] = i_ref[...]

    @pl.when(step == pl.num_programs(0) - 1)
    def _wait_cat():
        pltpu.make_async_copy(c_ref, oc_ref, csem).wait()


def kernel(emb_user, emb_item, emb_category):
    n, d = emb_user.shape
    grid = (n // _BLOCK,)
    big_spec = pl.BlockSpec((_BLOCK, d), lambda i: (i, 0))
    outs = pl.pallas_call(
        _copy_kernel,
        grid=grid,
        out_shape=tuple(
            jax.ShapeDtypeStruct(x.shape, x.dtype)
            for x in (emb_user, emb_item, emb_category)
        ),
        in_specs=[big_spec, big_spec, pl.BlockSpec(memory_space=pl.ANY)],
        out_specs=[big_spec, big_spec, pl.BlockSpec(memory_space=pl.ANY)],
        scratch_shapes=[pltpu.SemaphoreType.DMA],
    )(emb_user, emb_item, emb_category)
    return outs
